# baseline (speedup 1.0000x reference)
import functools

import jax
import jax.numpy as jnp
from jax import lax
from jax.experimental import pallas as pl
from jax.experimental.pallas import tpu as pltpu
from jax.experimental.pallas import tpu_sc as plsc

_NC = 2
_NS = 16
_NW = _NC * _NS
_CE = 4096
_SL = _CE // _NS
_UN = 4


def _make_sc_kernel(N, D, E_pad, N_pad):
    CPT = D // _NW
    CHUNKS = E_pad // _CE
    GROUPS = _CE // 16
    mesh = plsc.VectorSubcoreMesh(core_axis_name="c", subcore_axis_name="s")

    @functools.partial(
        pl.kernel,
        mesh=mesh,
        compiler_params=pltpu.CompilerParams(needs_layout_passes=False),
        out_type=(
            jax.ShapeDtypeStruct((D, N_pad), jnp.float32),
            jax.ShapeDtypeStruct((_NW * N_pad,), jnp.float32),
        ),
        scratch_types=[
            [pltpu.VMEM((N_pad,), jnp.float32) for _ in range(CPT)],
            [pltpu.VMEM((N_pad,), jnp.float32) for _ in range(CPT)],
            pltpu.VMEM((N_pad,), jnp.float32),
            pltpu.VMEM((2, _CE), jnp.int32),
            pltpu.VMEM((2, _CE), jnp.int32),
            pltpu.VMEM((2, _SL), jnp.int32),
            pltpu.VMEM((2, _SL), jnp.int32),
            pltpu.VMEM_SHARED((2, _CE), jnp.int32),
            pltpu.VMEM_SHARED((2, _CE), jnp.int32),
            pltpu.SemaphoreType.DMA((2,)),
            pltpu.SemaphoreType.DMA((2,)),
            pltpu.SemaphoreType.DMA((2,)),
            pltpu.SemaphoreType.DMA((4,)),
        ],
    )
    def sc(xt_hbm, src_hbm, dst_hbm, sums_hbm, cnts_hbm,
           xc, ac, cnt_loc, lsrc, ldst, bsrc, bdst, ssrc, sdst,
           sem_h, sem_p, sem_b, sem_x):
        c = lax.axis_index("c")
        s = lax.axis_index("s")
        w = c * _NS + s
        t = w
        zv = jnp.zeros((16,), jnp.float32)
        ones = jnp.ones((16,), jnp.float32)

        xcps = [
            pltpu.async_copy(xt_hbm.at[t * CPT + q], xc[q], sem_x.at[q])
            for q in range(CPT)
        ]

        def za_body(i, _):
            for q in range(CPT):
                ac[q][pl.ds(i * 16, 16)] = zv
            cnt_loc[pl.ds(i * 16, 16)] = zv
            return 0
        lax.fori_loop(0, N_pad // 16, za_body, 0)

        def stage(k, buf):
            h0 = pltpu.async_copy(
                src_hbm.at[k * _NS + s], bsrc.at[buf], sem_h.at[0])
            h1 = pltpu.async_copy(
                dst_hbm.at[k * _NS + s], bdst.at[buf], sem_h.at[1])
            h0.wait()
            h1.wait()
            p0 = pltpu.async_copy(
                bsrc.at[buf], ssrc.at[buf, pl.ds(s * _SL, _SL)], sem_p.at[0])
            p1 = pltpu.async_copy(
                bdst.at[buf], sdst.at[buf, pl.ds(s * _SL, _SL)], sem_p.at[1])
            p0.wait()
            p1.wait()

        for cp in xcps:
            cp.wait()
        stage(0, 0)
        plsc.subcore_barrier()

        def chunk_body(k, _):
            buf = lax.rem(k, 2)
            kn = jnp.minimum(k + 1, CHUNKS - 1)
            stage(kn, 1 - buf)
            b0 = pltpu.async_copy(ssrc.at[buf], lsrc.at[buf], sem_b.at[0])
            b1 = pltpu.async_copy(sdst.at[buf], ldst.at[buf], sem_b.at[1])
            b0.wait()
            b1.wait()

            def group_body(g, _):
                for u in range(_UN):
                    off = (g * _UN + u) * 16
                    vsrc = lsrc[buf, pl.ds(off, 16)]
                    vdst = ldst[buf, pl.ds(off, 16)]
                    for q in range(CPT):
                        vals = plsc.load_gather(xc[q], [vsrc])
                        plsc.addupdate_scatter(ac[q], [vdst], vals)
                    plsc.addupdate_scatter(cnt_loc, [vdst], ones)
                return 0
            lax.fori_loop(0, GROUPS // _UN, group_body, 0)
            plsc.subcore_barrier()
            return 0
        lax.fori_loop(0, CHUNKS, chunk_body, 0)

        wcps = [
            pltpu.async_copy(ac[q], sums_hbm.at[t * CPT + q], sem_x.at[q])
            for q in range(CPT)
        ]
        for cp in wcps:
            cp.wait()
        pltpu.sync_copy(cnt_loc, cnts_hbm.at[pl.ds(w * N_pad, N_pad)])

    return sc


def _tc_body(sums_ref, cnts_ref, w_ref, x_ref, o_ref):
    mm = lax.dot_general(sums_ref[...], w_ref[...],
                         dimension_numbers=(((0,), (1,)), ((), ())),
                         preferred_element_type=jnp.float32)
    cnt = jnp.sum(cnts_ref[...], axis=0) * (1.0 / _NW)
    o_ref[...] = mm / jnp.maximum(cnt, 1.0)[:, None] + x_ref[...]


def kernel(x, edge_index, W_rel):
    N, D = x.shape
    E = edge_index.shape[1]
    E_pad = -(-E // _CE) * _CE
    N_pad = -(-(N + 1) // (_NS * 16)) * (_NS * 16)

    src = edge_index[0]
    dst = edge_index[1]
    pad = E_pad - E
    if pad:
        src = jnp.concatenate([src, jnp.zeros((pad,), jnp.int32)])
        dst = jnp.concatenate([dst, jnp.full((pad,), N, jnp.int32)])
    src2 = src.reshape(E_pad // _SL, _SL)
    dst2 = dst.reshape(E_pad // _SL, _SL)
    xt = jnp.pad(x, ((0, N_pad - N), (0, 0))).T

    sums, cnts = _make_sc_kernel(N, D, E_pad, N_pad)(xt, src2, dst2)
    cnts = cnts.reshape(_NW, N_pad)

    BR = 1024
    NB = N_pad // BR
    out = pl.pallas_call(
        _tc_body,
        grid=(NB,),
        in_specs=[
            pl.BlockSpec((D, BR), lambda i: (0, i)),
            pl.BlockSpec((_NW, BR), lambda i: (0, i)),
            pl.BlockSpec((D, D), lambda i: (0, 0)),
            pl.BlockSpec((BR, D), lambda i: (i, 0)),
        ],
        out_specs=pl.BlockSpec((BR, D), lambda i: (i, 0)),
        out_shape=jax.ShapeDtypeStruct((N, D), jnp.float32),
    )(sums, cnts, W_rel, x)
    return out


# speedup vs baseline: 1.6002x; 1.6002x over previous
import functools

import jax
import jax.numpy as jnp
from jax import lax
from jax.experimental import pallas as pl
from jax.experimental.pallas import tpu as pltpu
from jax.experimental.pallas import tpu_sc as plsc

_NC = 2
_NS = 16
_NW = _NC * _NS
_CE = 8192
_SL = _CE // _NS
_UN = 8


def _make_sc_kernel(N, D, E_pad, N_pad):
    CPT = D // _NW
    CHUNKS = E_pad // _CE
    GROUPS = _CE // 16
    mesh = plsc.VectorSubcoreMesh(core_axis_name="c", subcore_axis_name="s")

    @functools.partial(
        pl.kernel,
        mesh=mesh,
        compiler_params=pltpu.CompilerParams(needs_layout_passes=False),
        out_type=(
            jax.ShapeDtypeStruct((D, N_pad), jnp.float32),
            jax.ShapeDtypeStruct((_NW * N_pad,), jnp.float32),
        ),
        scratch_types=[
            [pltpu.VMEM((N_pad,), jnp.float32) for _ in range(CPT)],
            [pltpu.VMEM((N_pad,), jnp.float32) for _ in range(CPT)],
            pltpu.VMEM((N_pad,), jnp.float32),
            [pltpu.VMEM((_CE,), jnp.int32) for _ in range(2)],
            [pltpu.VMEM((_CE,), jnp.int32) for _ in range(2)],
            pltpu.VMEM((2, _SL), jnp.int32),
            pltpu.VMEM((2, _SL), jnp.int32),
            pltpu.VMEM_SHARED((3, _CE), jnp.int32),
            pltpu.VMEM_SHARED((3, _CE), jnp.int32),
            pltpu.SemaphoreType.DMA((2,)),
            pltpu.SemaphoreType.DMA((2,)),
            pltpu.SemaphoreType.DMA((2,)),
            pltpu.SemaphoreType.DMA((4,)),
        ],
    )
    def sc(xt_hbm, src_hbm, dst_hbm, sums_hbm, cnts_hbm,
           xc, ac, cnt_loc, lsrc, ldst, bsrc, bdst, ssrc, sdst,
           sem_h, sem_p, sem_b, sem_x):
        c = lax.axis_index("c")
        s = lax.axis_index("s")
        w = c * _NS + s
        zv = jnp.zeros((16,), jnp.float32)
        ones = jnp.ones((16,), jnp.float32)

        xcps = [
            pltpu.async_copy(xt_hbm.at[w * CPT + q], xc[q], sem_x.at[q])
            for q in range(CPT)
        ]

        def za_body(i, _):
            for q in range(CPT):
                ac[q][pl.ds(i * 16, 16)] = zv
            cnt_loc[pl.ds(i * 16, 16)] = zv
            return 0
        lax.fori_loop(0, N_pad // 16, za_body, 0)

        def stage_sync(k, sb, bb):
            pltpu.async_copy(
                src_hbm.at[k * _NS + s], bsrc.at[bb], sem_h.at[0]).wait()
            pltpu.async_copy(
                dst_hbm.at[k * _NS + s], bdst.at[bb], sem_h.at[1]).wait()
            pltpu.async_copy(
                bsrc.at[bb], ssrc.at[sb, pl.ds(s * _SL, _SL)],
                sem_p.at[0]).wait()
            pltpu.async_copy(
                bdst.at[bb], sdst.at[sb, pl.ds(s * _SL, _SL)],
                sem_p.at[1]).wait()

        for cp in xcps:
            cp.wait()
        stage_sync(0, 0, 0)
        stage_sync(jnp.minimum(1, CHUNKS - 1), 1, 1)
        plsc.subcore_barrier()
        pltpu.async_copy(ssrc.at[0], lsrc[0], sem_b.at[0]).wait()
        pltpu.async_copy(sdst.at[0], ldst[0], sem_b.at[1]).wait()

        def compute(ls, ld, g0):
            for u in range(_UN):
                off = (g0 + u) * 16
                vsrc = ls[pl.ds(off, 16)]
                vdst = ld[pl.ds(off, 16)]
                vals = [plsc.load_gather(xc[q], [vsrc]) for q in range(CPT)]
                for q in range(CPT):
                    plsc.addupdate_scatter(ac[q], [vdst], vals[q])
                plsc.addupdate_scatter(cnt_loc, [vdst], ones)

        def one_chunk(k, b):
            sb1 = lax.rem(k + 1, 3)
            bc0 = pltpu.async_copy(ssrc.at[sb1], lsrc[1 - b], sem_b.at[0])
            bc1 = pltpu.async_copy(sdst.at[sb1], ldst[1 - b], sem_b.at[1])
            kn = jnp.minimum(k + 2, CHUNKS - 1)
            h0 = pltpu.async_copy(
                src_hbm.at[kn * _NS + s], bsrc.at[b], sem_h.at[0])
            h1 = pltpu.async_copy(
                dst_hbm.at[kn * _NS + s], bdst.at[b], sem_h.at[1])

            def group_body(g, _):
                compute(lsrc[b], ldst[b], g * _UN)
                return 0
            lax.fori_loop(0, GROUPS // _UN, group_body, 0)

            sb2 = lax.rem(k + 2, 3)
            h0.wait()
            h1.wait()
            pltpu.async_copy(
                bsrc.at[b], ssrc.at[sb2, pl.ds(s * _SL, _SL)],
                sem_p.at[0]).wait()
            pltpu.async_copy(
                bdst.at[b], sdst.at[sb2, pl.ds(s * _SL, _SL)],
                sem_p.at[1]).wait()
            bc0.wait()
            bc1.wait()
            plsc.subcore_barrier()

        def pair_body(i, _):
            one_chunk(2 * i, 0)
            one_chunk(2 * i + 1, 1)
            return 0
        lax.fori_loop(0, CHUNKS // 2, pair_body, 0)

        wcps = [
            pltpu.async_copy(ac[q], sums_hbm.at[w * CPT + q], sem_x.at[q])
            for q in range(CPT)
        ]
        for cp in wcps:
            cp.wait()
        pltpu.sync_copy(cnt_loc, cnts_hbm.at[pl.ds(w * N_pad, N_pad)])

    return sc


def _tc_body(sums_ref, cnts_ref, w_ref, x_ref, o_ref):
    mm = lax.dot_general(sums_ref[...], w_ref[...],
                         dimension_numbers=(((0,), (1,)), ((), ())),
                         preferred_element_type=jnp.float32)
    cnt = jnp.sum(cnts_ref[...], axis=0) * (1.0 / _NW)
    o_ref[...] = mm / jnp.maximum(cnt, 1.0)[:, None] + x_ref[...]


def kernel(x, edge_index, W_rel):
    N, D = x.shape
    E = edge_index.shape[1]
    E_pad = -(-E // (2 * _CE)) * (2 * _CE)
    N_pad = -(-(N + 1) // (_NS * 16)) * (_NS * 16)

    src = edge_index[0]
    dst = edge_index[1]
    pad = E_pad - E
    if pad:
        src = jnp.concatenate([src, jnp.zeros((pad,), jnp.int32)])
        dst = jnp.concatenate([dst, jnp.full((pad,), N, jnp.int32)])
    src2 = src.reshape(E_pad // _SL, _SL)
    dst2 = dst.reshape(E_pad // _SL, _SL)
    xt = jnp.pad(x, ((0, N_pad - N), (0, 0))).T

    sums, cnts = _make_sc_kernel(N, D, E_pad, N_pad)(xt, src2, dst2)
    cnts = cnts.reshape(_NW, N_pad)

    BR = 1024
    NB = N_pad // BR
    out = pl.pallas_call(
        _tc_body,
        grid=(NB,),
        in_specs=[
            pl.BlockSpec((D, BR), lambda i: (0, i)),
            pl.BlockSpec((_NW, BR), lambda i: (0, i)),
            pl.BlockSpec((D, D), lambda i: (0, 0)),
            pl.BlockSpec((BR, D), lambda i: (i, 0)),
        ],
        out_specs=pl.BlockSpec((BR, D), lambda i: (i, 0)),
        out_shape=jax.ShapeDtypeStruct((N, D), jnp.float32),
    )(sums, cnts, W_rel, x)
    return out
